# baseline (speedup 1.0000x reference)
import functools

import jax
import jax.numpy as jnp
from jax import lax
from jax.experimental import pallas as pl
from jax.experimental.pallas import tpu as pltpu
from jax.experimental.pallas import tpu_sc as plsc

NS = 16
L = 16
OWN_BITS = 16
OWN = 1 << OWN_BITS


def _dyemb_sc(batch, dimp):
    rows_t = batch // NS
    nvec = batch // L
    cpt = nvec // NS

    mesh = plsc.VectorSubcoreMesh(
        core_axis_name="c", subcore_axis_name="s", num_cores=1)

    @functools.partial(
        pl.kernel,
        out_type=jax.ShapeDtypeStruct((batch, dimp), jnp.float32),
        mesh=mesh,
        compiler_params=pltpu.CompilerParams(
            needs_layout_passes=False, use_tc_tiling_on_sc=True),
        scratch_types=[
            pltpu.HBM((NS * OWN,), jnp.int32),
            pltpu.VMEM_SHARED((batch,), jnp.int32),
            pltpu.VMEM((batch,), jnp.int32),
            pltpu.VMEM((OWN,), jnp.int32),
            pltpu.VMEM((batch,), jnp.int32),
            pltpu.VMEM((cpt * L,), jnp.int32),
            pltpu.VMEM((rows_t,), jnp.int32),
            pltpu.VMEM((2, 64, dimp), jnp.float32),
            pltpu.SemaphoreType.DMA,
        ],
    )
    def k(idx_hbm, values_hbm, out_hbm, p_tab, masks_sp, idx_v, tab_v,
          masks_v, mbuf_v, p_v, rows_v, sem):
        tid = lax.axis_index("s")
        lane = lax.iota(jnp.int32, L)

        pltpu.sync_copy(idx_hbm, idx_v)

        def mask_step(j, carry):
            start = pl.multiple_of((tid * cpt + j) * L, L)
            x = idx_v[pl.ds(start, L)]
            _, last = plsc.scan_count(x)
            jstart = pl.multiple_of(j * L, L)
            mbuf_v[pl.ds(jstart, L)] = last.astype(jnp.int32)
            return carry

        lax.fori_loop(0, cpt, mask_step, 0, unroll=4)
        pltpu.sync_copy(mbuf_v, masks_sp.at[pl.ds(tid * cpt * L, cpt * L)])
        plsc.subcore_barrier()
        pltpu.sync_copy(masks_sp, masks_v)

        def scan_step(i, carry):
            start = pl.multiple_of(i * L, L)
            x = idx_v[pl.ds(start, L)]
            last = masks_v[pl.ds(start, L)] == 1
            mine = lax.shift_right_logical(x, OWN_BITS) == tid
            xl = x & (OWN - 1)
            pos = i * L + lane
            plsc.store_scatter(tab_v, [xl], pos, mask=last & mine)
            return carry

        lax.fori_loop(0, nvec, scan_step, 0, unroll=8)

        pltpu.sync_copy(tab_v, p_tab.at[pl.ds(tid * OWN, OWN)])
        plsc.subcore_barrier()

        tbase = tid * rows_t
        cps = [
            pltpu.async_copy(
                p_tab.at[idx_v.at[pl.ds(tbase + c * 128, 128)]],
                p_v.at[pl.ds(c * 128, 128)], sem)
            for c in range(rows_t // 128)
        ]
        for cp in cps:
            cp.wait()

        def row_gather(c, buf):
            return pltpu.async_copy(
                values_hbm.at[p_v.at[pl.ds(c * 64, 64)]],
                rows_v.at[buf], sem)
        rchunks = rows_t // 64
        pend = row_gather(0, 0)
        for c in range(rchunks):
            pend.wait()
            if c + 1 < rchunks:
                nxt = row_gather(c + 1, (c + 1) % 2)
            pltpu.sync_copy(rows_v.at[c % 2],
                            out_hbm.at[pl.ds(tbase + c * 64, 64)])
            if c + 1 < rchunks:
                pend = nxt

    return k


@jax.jit
def kernel(raw_feature, node_idxs, values):
    del raw_feature
    batch, dim = values.shape
    values128 = jnp.pad(values, ((0, 0), (0, 128 - dim)))
    out128 = _dyemb_sc(batch, 128)(node_idxs.astype(jnp.int32), values128)
    return out128[:, :dim]


# speedup vs baseline: 1.0007x; 1.0007x over previous
import functools

import jax
import jax.numpy as jnp
from jax import lax
from jax.experimental import pallas as pl
from jax.experimental.pallas import tpu as pltpu
from jax.experimental.pallas import tpu_sc as plsc

NS = 16
L = 16
OWN_BITS = 16
OWN = 1 << OWN_BITS


def _dyemb_sc(batch, dimp):
    rows_t = batch // NS
    nvec = batch // L
    cpt = nvec // NS

    mesh = plsc.VectorSubcoreMesh(
        core_axis_name="c", subcore_axis_name="s", num_cores=1)

    @functools.partial(
        pl.kernel,
        out_type=jax.ShapeDtypeStruct((batch, dimp), jnp.float32),
        mesh=mesh,
        compiler_params=pltpu.CompilerParams(
            needs_layout_passes=False, use_tc_tiling_on_sc=True),
        scratch_types=[
            pltpu.HBM((NS * OWN,), jnp.int32),
            pltpu.VMEM_SHARED((batch,), jnp.int32),
            pltpu.VMEM((batch,), jnp.int32),
            pltpu.VMEM((OWN,), jnp.int32),
            pltpu.VMEM((batch,), jnp.int32),
            pltpu.VMEM((cpt * L,), jnp.int32),
            pltpu.VMEM((rows_t,), jnp.int32),
            pltpu.VMEM((2, 64, dimp), jnp.float32),
            pltpu.SemaphoreType.DMA,
        ],
    )
    def k(idx_hbm, values_hbm, out_hbm, p_tab, masks_sp, idx_v, tab_v,
          masks_v, mbuf_v, p_v, rows_v, sem):
        tid = lax.axis_index("s")
        lane = lax.iota(jnp.int32, L)

        with jax.named_scope("idx_stage"):
            pltpu.sync_copy(idx_hbm, idx_v)

        def mask_step(j, carry):
            start = pl.multiple_of((tid * cpt + j) * L, L)
            x = idx_v[pl.ds(start, L)]
            _, last = plsc.scan_count(x)
            jstart = pl.multiple_of(j * L, L)
            mbuf_v[pl.ds(jstart, L)] = last.astype(jnp.int32)
            return carry

        with jax.named_scope("mask_phase"):
            lax.fori_loop(0, cpt, mask_step, 0, unroll=4)
            pltpu.sync_copy(mbuf_v, masks_sp.at[pl.ds(tid * cpt * L, cpt * L)])
            plsc.subcore_barrier()
            pltpu.sync_copy(masks_sp, masks_v)

        def scan_step(i, carry):
            start = pl.multiple_of(i * L, L)
            x = idx_v[pl.ds(start, L)]
            last = masks_v[pl.ds(start, L)] == 1
            mine = lax.shift_right_logical(x, OWN_BITS) == tid
            xl = x & (OWN - 1)
            pos = i * L + lane
            plsc.store_scatter(tab_v, [xl], pos, mask=last & mine)
            return carry

        with jax.named_scope("scan_phase"):
            lax.fori_loop(0, nvec, scan_step, 0, unroll=8)

        with jax.named_scope("publish_phase"):
            pltpu.sync_copy(tab_v, p_tab.at[pl.ds(tid * OWN, OWN)])
            plsc.subcore_barrier()

        tbase = tid * rows_t
        with jax.named_scope("winner_gather"):
            cps = [
                pltpu.async_copy(
                    p_tab.at[idx_v.at[pl.ds(tbase + c * 128, 128)]],
                    p_v.at[pl.ds(c * 128, 128)], sem)
                for c in range(rows_t // 128)
            ]
            for cp in cps:
                cp.wait()

        def row_gather(c, buf):
            return pltpu.async_copy(
                values_hbm.at[p_v.at[pl.ds(c * 64, 64)]],
                rows_v.at[buf], sem)
        rchunks = rows_t // 64
        with jax.named_scope("row_emit"):
            pend = row_gather(0, 0)
            for c in range(rchunks):
                pend.wait()
                if c + 1 < rchunks:
                    nxt = row_gather(c + 1, (c + 1) % 2)
                pltpu.sync_copy(rows_v.at[c % 2],
                                out_hbm.at[pl.ds(tbase + c * 64, 64)])
                if c + 1 < rchunks:
                    pend = nxt

    return k


@jax.jit
def kernel(raw_feature, node_idxs, values):
    del raw_feature
    batch, dim = values.shape
    values128 = jnp.pad(values, ((0, 0), (0, 128 - dim)))
    out128 = _dyemb_sc(batch, 128)(node_idxs.astype(jnp.int32), values128)
    return out128[:, :dim]


# speedup vs baseline: 1.1374x; 1.1366x over previous
import functools

import jax
import jax.numpy as jnp
from jax import lax
from jax.experimental import pallas as pl
from jax.experimental.pallas import tpu as pltpu
from jax.experimental.pallas import tpu_sc as plsc

NS = 16
L = 16
OWN_BITS = 16
OWN = 1 << OWN_BITS


def _dyemb_sc(batch, dimp):
    rows_t = batch // NS
    nvec = batch // L
    cpt = nvec // NS

    mesh = plsc.VectorSubcoreMesh(
        core_axis_name="c", subcore_axis_name="s", num_cores=1)

    @functools.partial(
        pl.kernel,
        out_type=jax.ShapeDtypeStruct((batch, dimp), jnp.float32),
        mesh=mesh,
        compiler_params=pltpu.CompilerParams(
            needs_layout_passes=False, use_tc_tiling_on_sc=True),
        scratch_types=[
            pltpu.HBM((NS * OWN,), jnp.int32),
            pltpu.VMEM_SHARED((batch,), jnp.int32),
            pltpu.VMEM((batch,), jnp.int32),
            pltpu.VMEM((OWN,), jnp.int32),
            pltpu.VMEM((batch,), jnp.int32),
            pltpu.VMEM((cpt * L,), jnp.int32),
            pltpu.VMEM((rows_t,), jnp.int32),
            pltpu.VMEM((3, 64, dimp), jnp.float32),
            pltpu.SemaphoreType.DMA,
            pltpu.SemaphoreType.DMA((3,)),
            pltpu.SemaphoreType.DMA((3,)),
        ],
    )
    def k(idx_hbm, values_hbm, out_hbm, p_tab, masks_sp, idx_v, tab_v,
          masks_v, mbuf_v, p_v, rows_v, sem, gsem, osem):
        tid = lax.axis_index("s")
        lane = lax.iota(jnp.int32, L)

        with jax.named_scope("idx_stage"):
            pltpu.sync_copy(idx_hbm, idx_v)

        def mask_step(j, carry):
            start = pl.multiple_of((tid * cpt + j) * L, L)
            x = idx_v[pl.ds(start, L)]
            _, last = plsc.scan_count(x)
            jstart = pl.multiple_of(j * L, L)
            mbuf_v[pl.ds(jstart, L)] = last.astype(jnp.int32)
            return carry

        with jax.named_scope("mask_phase"):
            lax.fori_loop(0, cpt, mask_step, 0, unroll=4)
            pltpu.sync_copy(mbuf_v, masks_sp.at[pl.ds(tid * cpt * L, cpt * L)])
            plsc.subcore_barrier()
            pltpu.sync_copy(masks_sp, masks_v)

        def scan_step(i, carry):
            start = pl.multiple_of(i * L, L)
            x = idx_v[pl.ds(start, L)]
            last = masks_v[pl.ds(start, L)] == 1
            mine = lax.shift_right_logical(x, OWN_BITS) == tid
            xl = x & (OWN - 1)
            pos = i * L + lane
            plsc.store_scatter(tab_v, [xl], pos, mask=last & mine)
            return carry

        with jax.named_scope("scan_phase"):
            lax.fori_loop(0, nvec, scan_step, 0, unroll=8)

        with jax.named_scope("publish_phase"):
            pltpu.sync_copy(tab_v, p_tab.at[pl.ds(tid * OWN, OWN)])
            plsc.subcore_barrier()

        tbase = tid * rows_t
        with jax.named_scope("winner_gather"):
            cps = [
                pltpu.async_copy(
                    p_tab.at[idx_v.at[pl.ds(tbase + c * 128, 128)]],
                    p_v.at[pl.ds(c * 128, 128)], sem)
                for c in range(rows_t // 128)
            ]
            for cp in cps:
                cp.wait()

        depth = 3
        rchunks = rows_t // 64

        def row_gather(c, b):
            return pltpu.async_copy(
                values_hbm.at[p_v.at[pl.ds(c * 64, 64)]],
                rows_v.at[b], gsem.at[b])

        with jax.named_scope("row_emit"):
            pend_g = [row_gather(b, b) for b in range(depth)]
            pend_o = [None] * rchunks
            for c in range(rchunks):
                b = c % depth
                pend_g[b].wait()
                pend_o[c] = pltpu.async_copy(
                    rows_v.at[b], out_hbm.at[pl.ds(tbase + c * 64, 64)],
                    osem.at[b])
                if c + depth < rchunks:
                    pend_o[c].wait()
                    pend_g[b] = row_gather(c + depth, b)
            for c in range(max(0, rchunks - depth), rchunks):
                pend_o[c].wait()

    return k


@jax.jit
def kernel(raw_feature, node_idxs, values):
    del raw_feature
    batch, dim = values.shape
    values128 = jnp.pad(values, ((0, 0), (0, 128 - dim)))
    out128 = _dyemb_sc(batch, 128)(node_idxs.astype(jnp.int32), values128)
    return out128[:, :dim]


# speedup vs baseline: 1.1582x; 1.0183x over previous
import functools

import jax
import jax.numpy as jnp
from jax import lax
from jax.experimental import pallas as pl
from jax.experimental.pallas import tpu as pltpu
from jax.experimental.pallas import tpu_sc as plsc

NS = 16
L = 16
OWN_BITS = 16
OWN = 1 << OWN_BITS


def _dyemb_sc(batch, dimp):
    rows_t = batch // NS
    nvec = batch // L
    cpt = nvec // NS

    mesh = plsc.VectorSubcoreMesh(
        core_axis_name="c", subcore_axis_name="s", num_cores=1)

    @functools.partial(
        pl.kernel,
        out_type=jax.ShapeDtypeStruct((batch, dimp), jnp.float32),
        mesh=mesh,
        compiler_params=pltpu.CompilerParams(
            needs_layout_passes=False, use_tc_tiling_on_sc=True),
        scratch_types=[
            pltpu.HBM((NS * OWN,), jnp.int32),
            pltpu.VMEM_SHARED((batch,), jnp.int32),
            pltpu.VMEM((batch,), jnp.int32),
            pltpu.VMEM((OWN,), jnp.int32),
            pltpu.VMEM((cpt * L,), jnp.int32),
            pltpu.VMEM((rows_t,), jnp.int32),
            pltpu.VMEM((5, 64, dimp), jnp.float32),
            pltpu.SemaphoreType.DMA,
            pltpu.SemaphoreType.DMA((5,)),
            pltpu.SemaphoreType.DMA((5,)),
        ],
    )
    def k(idx_hbm, values_hbm, out_hbm, p_tab, masks_sp, idx_v, tab_v,
          mbuf_v, p_v, rows_v, sem, gsem, osem):
        tid = lax.axis_index("s")
        lane = lax.iota(jnp.int32, L)

        with jax.named_scope("idx_stage"):
            pltpu.sync_copy(idx_hbm, idx_v)

        sign = jnp.int32(-2147483648)

        def mask_step(j, carry):
            start = pl.multiple_of((tid * cpt + j) * L, L)
            x = idx_v[pl.ds(start, L)]
            _, last = plsc.scan_count(x)
            jstart = pl.multiple_of(j * L, L)
            mbuf_v[pl.ds(jstart, L)] = jnp.where(last, x | sign, x)
            return carry

        with jax.named_scope("mask_phase"):
            lax.fori_loop(0, cpt, mask_step, 0, unroll=4)
            pltpu.sync_copy(mbuf_v, masks_sp.at[pl.ds(tid * cpt * L, cpt * L)])
            plsc.subcore_barrier()
            pltpu.sync_copy(masks_sp, idx_v)

        def scan_step(i, carry):
            start = pl.multiple_of(i * L, L)
            x = idx_v[pl.ds(start, L)]
            last = x < 0
            mine = (lax.shift_right_logical(x, OWN_BITS) & (NS - 1)) == tid
            xl = x & (OWN - 1)
            pos = i * L + lane
            plsc.store_scatter(tab_v, [xl], pos, mask=last & mine)
            return carry

        with jax.named_scope("scan_phase"):
            lax.fori_loop(0, nvec, scan_step, 0, unroll=8)

        with jax.named_scope("publish_phase"):
            pltpu.sync_copy(tab_v, p_tab.at[pl.ds(tid * OWN, OWN)])
            plsc.subcore_barrier()

        tbase = tid * rows_t

        def clean_step(j, carry):
            start = pl.multiple_of(j * L, L)
            mbuf_v[pl.ds(start, L)] = (
                idx_v[pl.ds(pl.multiple_of(tbase + j * L, L), L)] & ~sign)
            return carry

        with jax.named_scope("winner_gather"):
            lax.fori_loop(0, rows_t // L, clean_step, 0, unroll=8)
            cps = [
                pltpu.async_copy(
                    p_tab.at[mbuf_v.at[pl.ds(c * 128, 128)]],
                    p_v.at[pl.ds(c * 128, 128)], sem)
                for c in range(rows_t // 128)
            ]
            for cp in cps:
                cp.wait()

        depth = 5
        rchunks = rows_t // 64

        def row_gather(c, b):
            return pltpu.async_copy(
                values_hbm.at[p_v.at[pl.ds(c * 64, 64)]],
                rows_v.at[b], gsem.at[b])

        with jax.named_scope("row_emit"):
            pend_g = [row_gather(b, b) for b in range(depth)]
            pend_o = [None] * rchunks
            for c in range(rchunks):
                b = c % depth
                pend_g[b].wait()
                pend_o[c] = pltpu.async_copy(
                    rows_v.at[b], out_hbm.at[pl.ds(tbase + c * 64, 64)],
                    osem.at[b])
                if c + depth < rchunks:
                    pend_o[c].wait()
                    pend_g[b] = row_gather(c + depth, b)
            for c in range(max(0, rchunks - depth), rchunks):
                pend_o[c].wait()

    return k


@jax.jit
def kernel(raw_feature, node_idxs, values):
    del raw_feature
    batch, dim = values.shape
    values128 = jnp.pad(values, ((0, 0), (0, 128 - dim)))
    out128 = _dyemb_sc(batch, 128)(node_idxs.astype(jnp.int32), values128)
    return out128[:, :dim]
